# Initial kernel scaffold; baseline (speedup 1.0000x reference)
import jax
import jax.numpy as jnp
from jax.experimental import pallas as pl


def kernel(view0, W1, b1, W2, b2, Wr, br, expert_keys):
    raise NotImplementedError("write your pallas kernel here")


# speedup vs baseline: 4.3874x; 4.3874x over previous
import functools

import jax
import jax.numpy as jnp
from jax.experimental import pallas as pl
from jax.experimental.pallas import tpu as pltpu

B, L, D, E, K = 1, 2048, 768, 8, 2
FT = 768
NF = (4 * D) // FT


def _moe_kernel(h_ref, w1_ref, b1_ref, w2_ref, b2_ref, wr_ref, br_ref,
                keys_ref, out_ref, cmat_ref):
    e = pl.program_id(0)
    f = pl.program_id(1)
    first = jnp.logical_and(e == 0, f == 0)

    @pl.when(first)
    def _router():
        h = h_ref[...]
        ek = keys_ref[...]
        hn = jnp.sum(h * h, axis=1, keepdims=True)
        kn = jnp.sum(ek * ek, axis=1)[None, :]
        cross = jax.lax.dot_general(h, ek, (((1,), (1,)), ((), ())),
                                    preferred_element_type=jnp.float32)
        sq = hn + kn - 2.0 * cross
        dist = jnp.sqrt(jnp.maximum(sq, 0.0))
        logits = -dist * dist + jnp.dot(
            h, wr_ref[...], preferred_element_type=jnp.float32) + br_ref[...]
        iota = jax.lax.broadcasted_iota(jnp.int32, (L, E), 1)
        m1 = jnp.max(logits, axis=1, keepdims=True)
        i1 = jnp.argmax(logits, axis=1)[:, None]
        masked = jnp.where(iota == i1, -jnp.inf, logits)
        m2 = jnp.max(masked, axis=1, keepdims=True)
        i2 = jnp.argmax(masked, axis=1)[:, None]
        e2 = jnp.exp(m2 - m1)
        denom = 1.0 + e2
        w_a = 1.0 / denom
        w_b = e2 / denom
        cmat_ref[...] = (w_a * (iota == i1).astype(jnp.float32)
                         + w_b * (iota == i2).astype(jnp.float32))

    x = h_ref[...]
    pre = jnp.dot(x, w1_ref[0], preferred_element_type=jnp.float32) + b1_ref[0]
    hid = 0.5 * pre * (1.0 + jax.lax.erf(pre * 0.7071067811865476))
    part = jnp.dot(hid, w2_ref[0], preferred_element_type=jnp.float32)
    lane = jax.lax.broadcasted_iota(jnp.int32, (L, E), 1)
    coef = jnp.sum(jnp.where(lane == e, cmat_ref[...], 0.0),
                   axis=1, keepdims=True)

    part = part + jnp.where(f == 0, 1.0, 0.0) * b2_ref[0]
    contrib = coef * part

    @pl.when(first)
    def _init():
        out_ref[...] = contrib

    @pl.when(jnp.logical_not(first))
    def _acc():
        out_ref[...] += contrib


@functools.partial(jax.jit, static_argnames=())
def kernel(view0, W1, b1, W2, b2, Wr, br, expert_keys):
    h = view0.reshape(L, D)
    br2 = br.reshape(1, E)
    b1r = b1.reshape(E, 1, 4 * D)
    b2r = b2.reshape(E, 1, D)
    grid = (E, NF)
    out = pl.pallas_call(
        _moe_kernel,
        grid=grid,
        in_specs=[
            pl.BlockSpec((L, D), lambda e, f: (0, 0)),
            pl.BlockSpec((1, D, FT), lambda e, f: (e, 0, f)),
            pl.BlockSpec((1, 1, FT), lambda e, f: (e, 0, f)),
            pl.BlockSpec((1, FT, D), lambda e, f: (e, f, 0)),
            pl.BlockSpec((1, 1, D), lambda e, f: (e, 0, 0)),
            pl.BlockSpec((D, E), lambda e, f: (0, 0)),
            pl.BlockSpec((1, E), lambda e, f: (0, 0)),
            pl.BlockSpec((E, D), lambda e, f: (0, 0)),
        ],
        out_specs=pl.BlockSpec((L, D), lambda e, f: (0, 0)),
        out_shape=jax.ShapeDtypeStruct((L, D), jnp.float32),
        scratch_shapes=[pltpu.VMEM((L, E), jnp.float32)],
    )(h, W1, b1r, W2, b2r, Wr, br2, expert_keys)
    return out.reshape(B, L, D)
